# baseline (speedup 1.0000x reference)
import numpy as np

import jax
import jax.numpy as jnp
from jax.experimental import pallas as pl
from jax.experimental.pallas import tpu as pltpu

_N_ACT = 6
_BB = 32


def _w1_perm():
    perm = np.empty(256, np.int32)
    for c in range(4):
        for bi in range(2):
            for j in range(8):
                for r in range(4):
                    i = 4 * bi + r
                    perm[(c * 2 + bi) * 32 + j * 4 + r] = (i * 8 + j) * 4 + c
    return perm


_PERM1 = _w1_perm()


def _dqn_kernel(xs_ref, w1_ref, b1_ref, w2_ref, b2_ref, w3_ref, b3_ref,
                wl1_ref, bl1_ref, wl2_ref, bl2_ref, o_ref):
    bb = xs_ref.shape[0]
    x = xs_ref[...].reshape(bb, 4, 11, 2, 11, 32)

    def rowsel(c, off):
        if off < 2:
            return x[:, c, 0:10, off]
        return x[:, c, 1:11, 0]

    def colsel(xr, q2):
        if q2 == 0:
            return xr[:, :, 0:10, :]
        return jnp.concatenate([xr[:, :, 0:10, 16:], xr[:, :, 1:11, :16]],
                               axis=-1)

    m = [[None, None], [None, None]]
    for r2 in (0, 1):
        for q2 in (0, 1):
            p = jnp.concatenate(
                [colsel(rowsel(c, r2 + bi), q2)
                 for c in range(4) for bi in (0, 1)], axis=-1)
            a = jnp.dot(p.reshape(bb * 100, 256), w1_ref[...],
                        preferred_element_type=jnp.float32)
            a = jnp.maximum(a + b1_ref[...], 0.0).astype(jnp.bfloat16)
            m[r2][q2] = a.reshape(bb, 10, 10, 32)

    p = jnp.concatenate(
        [m[i % 2][j % 2][:, i // 2:i // 2 + 9, j // 2:j // 2 + 9, :]
         for i in range(4) for j in range(4)], axis=-1)
    a = jnp.dot(p.reshape(bb * 81, 512), w2_ref[...],
                preferred_element_type=jnp.float32)
    a = jnp.maximum(a + b2_ref[...], 0.0).astype(jnp.bfloat16)
    a = a.reshape(bb, 9, 9, 64)

    p = jnp.concatenate(
        [a[:, i:i + 7, j:j + 7, :] for i in range(3) for j in range(3)],
        axis=-1)
    a = jnp.dot(p.reshape(bb * 49, 576), w3_ref[...],
                preferred_element_type=jnp.float32)
    a = jnp.maximum(a + b3_ref[...], 0.0).astype(jnp.bfloat16)

    a = a.reshape(bb, 49, 64)
    flat = jnp.concatenate([a[:, p_, :] for p_ in range(49)], axis=-1)

    h = jnp.dot(flat, wl1_ref[...], preferred_element_type=jnp.float32)
    h = jnp.maximum(h + bl1_ref[...], 0.0).astype(jnp.bfloat16)
    q = jnp.dot(h, wl2_ref[...], preferred_element_type=jnp.float32)
    o_ref[...] = q + bl2_ref[...]


def kernel(w1, b1, w2, b2, w3, b3, wl1, bl1, wl2, bl2, x):
    B = x.shape[0]
    Bp = (B + _BB - 1) // _BB * _BB
    if Bp != B:
        x = jnp.pad(x, ((0, Bp - B), (0, 0), (0, 0), (0, 0)))

    xp = jnp.pad(x.astype(jnp.bfloat16), ((0, 0), (0, 0), (0, 4), (0, 4)))
    xs = jnp.transpose(xp.reshape(Bp, 4, 22, 4, 11, 8),
                       (0, 1, 2, 4, 5, 3)).reshape(Bp, 4, 22, 11, 32)
    w1p = w1[_PERM1]

    q = pl.pallas_call(
        _dqn_kernel,
        out_shape=jax.ShapeDtypeStruct((Bp, 128), jnp.float32),
        grid=(Bp // _BB,),
        in_specs=[
            pl.BlockSpec((_BB, 4, 22, 11, 32), lambda i: (i, 0, 0, 0, 0)),
            pl.BlockSpec((256, 32), lambda i: (0, 0)),
            pl.BlockSpec((1, 32), lambda i: (0, 0)),
            pl.BlockSpec((512, 64), lambda i: (0, 0)),
            pl.BlockSpec((1, 64), lambda i: (0, 0)),
            pl.BlockSpec((576, 64), lambda i: (0, 0)),
            pl.BlockSpec((1, 64), lambda i: (0, 0)),
            pl.BlockSpec((3136, 512), lambda i: (0, 0)),
            pl.BlockSpec((1, 512), lambda i: (0, 0)),
            pl.BlockSpec((512, 128), lambda i: (0, 0)),
            pl.BlockSpec((1, 128), lambda i: (0, 0)),
        ],
        out_specs=pl.BlockSpec((_BB, 128), lambda i: (i, 0)),
        compiler_params=pltpu.CompilerParams(
            dimension_semantics=("parallel",),
            vmem_limit_bytes=64 * 1024 * 1024,
        ),
    )(xs, w1p, b1.reshape(1, 32), w2, b2.reshape(1, 64), w3, b3.reshape(1, 64),
      wl1, bl1.reshape(1, 512), wl2, bl2.reshape(1, 128))
    return q[:B, :_N_ACT]


# speedup vs baseline: 1.6910x; 1.6910x over previous
import numpy as np

import jax
import jax.numpy as jnp
from jax.experimental import pallas as pl
from jax.experimental.pallas import tpu as pltpu

_N_ACT = 6
_BB = 64


def _w1_perm():
    perm = np.empty(256, np.int32)
    for bi in range(2):
        for j in range(8):
            for r in range(4):
                for c in range(4):
                    i = 4 * bi + r
                    perm[bi * 128 + j * 16 + r * 4 + c] = (i * 8 + j) * 4 + c
    return perm


_PERM1 = _w1_perm()


def _dqn_kernel(xs_ref, w1_ref, b1_ref, w2_ref, b2_ref, w3_ref, b3_ref,
                wl1_ref, bl1_ref, wl2_ref, bl2_ref, o_ref):
    bb = xs_ref.shape[0]
    x = xs_ref[...].reshape(bb, 11, 2, 11, 128)

    def rowsel(off):
        if off < 2:
            return x[:, 0:10, off]
        return x[:, 1:11, 0]

    def colsel(xr, q2):
        if q2 == 0:
            return xr[:, :, 0:10, :]
        return jnp.concatenate([xr[:, :, 0:10, 64:], xr[:, :, 1:11, :64]],
                               axis=-1)

    m = [[None, None], [None, None]]
    for r2 in (0, 1):
        for q2 in (0, 1):
            p = jnp.concatenate(
                [colsel(rowsel(r2 + bi), q2) for bi in (0, 1)], axis=-1)
            a = jnp.dot(p.reshape(bb * 100, 256), w1_ref[...],
                        preferred_element_type=jnp.float32)
            a = jnp.maximum(a + b1_ref[...], 0.0).astype(jnp.bfloat16)
            m[r2][q2] = a.reshape(bb, 10, 10, 32)

    p = jnp.concatenate(
        [m[i % 2][j % 2][:, i // 2:i // 2 + 9, j // 2:j // 2 + 9, :]
         for i in range(4) for j in range(4)], axis=-1)
    a = jnp.dot(p.reshape(bb * 81, 512), w2_ref[...],
                preferred_element_type=jnp.float32)
    a = jnp.maximum(a + b2_ref[...], 0.0).astype(jnp.bfloat16)
    a = a.reshape(bb, 9, 9, 64)

    p = jnp.concatenate(
        [a[:, i:i + 7, j:j + 7, :] for i in range(3) for j in range(3)],
        axis=-1)
    a = jnp.dot(p.reshape(bb * 49, 576), w3_ref[...],
                preferred_element_type=jnp.float32)
    a = jnp.maximum(a + b3_ref[...], 0.0).astype(jnp.bfloat16)

    a = a.reshape(bb, 49, 64)
    flat = jnp.concatenate([a[:, p, :] for p in range(49)], axis=-1)

    h = jnp.dot(flat, wl1_ref[...], preferred_element_type=jnp.float32)
    h = jnp.maximum(h + bl1_ref[...], 0.0).astype(jnp.bfloat16)
    q = jnp.dot(h, wl2_ref[...], preferred_element_type=jnp.float32)
    o_ref[...] = q + bl2_ref[...]


def kernel(w1, b1, w2, b2, w3, b3, wl1, bl1, wl2, bl2, x):
    B = x.shape[0]
    Bp = (B + _BB - 1) // _BB * _BB
    if Bp != B:
        x = jnp.pad(x, ((0, Bp - B), (0, 0), (0, 0), (0, 0)))

    xp = jnp.pad(x, ((0, 0), (0, 0), (0, 4), (0, 4)))
    xs = jnp.transpose(xp.reshape(Bp, 4, 22, 4, 11, 8),
                       (0, 2, 4, 5, 3, 1)).reshape(Bp, 22, 11, 128)
    xs = xs.astype(jnp.bfloat16)
    w1p = w1[_PERM1]

    q = pl.pallas_call(
        _dqn_kernel,
        out_shape=jax.ShapeDtypeStruct((Bp, 128), jnp.float32),
        grid=(Bp // _BB,),
        in_specs=[
            pl.BlockSpec((_BB, 22, 11, 128), lambda i: (i, 0, 0, 0)),
            pl.BlockSpec((256, 32), lambda i: (0, 0)),
            pl.BlockSpec((1, 32), lambda i: (0, 0)),
            pl.BlockSpec((512, 64), lambda i: (0, 0)),
            pl.BlockSpec((1, 64), lambda i: (0, 0)),
            pl.BlockSpec((576, 64), lambda i: (0, 0)),
            pl.BlockSpec((1, 64), lambda i: (0, 0)),
            pl.BlockSpec((3136, 512), lambda i: (0, 0)),
            pl.BlockSpec((1, 512), lambda i: (0, 0)),
            pl.BlockSpec((512, 128), lambda i: (0, 0)),
            pl.BlockSpec((1, 128), lambda i: (0, 0)),
        ],
        out_specs=pl.BlockSpec((_BB, 128), lambda i: (i, 0)),
        compiler_params=pltpu.CompilerParams(
            dimension_semantics=("parallel",),
            vmem_limit_bytes=64 * 1024 * 1024,
        ),
    )(xs, w1p, b1.reshape(1, 32), w2, b2.reshape(1, 64), w3, b3.reshape(1, 64),
      wl1, bl1.reshape(1, 512), wl2, bl2.reshape(1, 128))
    return q[:B, :_N_ACT]


# speedup vs baseline: 1.6930x; 1.0011x over previous
import numpy as np

import jax
import jax.numpy as jnp
from jax.experimental import pallas as pl
from jax.experimental.pallas import tpu as pltpu

_N_ACT = 6
_BB = 64


def _w1_perm():
    perm = np.empty(256, np.int32)
    for bi in range(2):
        for j in range(8):
            for r in range(4):
                for c in range(4):
                    i = 4 * bi + r
                    perm[bi * 128 + j * 16 + c * 4 + r] = (i * 8 + j) * 4 + c
    return perm


_PERM1 = _w1_perm()


def _dqn_kernel(xs_ref, w1_ref, b1_ref, w2_ref, b2_ref, w3_ref, b3_ref,
                wl1_ref, bl1_ref, wl2_ref, bl2_ref, o_ref):
    bb = xs_ref.shape[0]
    x = xs_ref[...].reshape(bb, 11, 2, 11, 128)

    def rowsel(off):
        if off < 2:
            return x[:, 0:10, off]
        return x[:, 1:11, 0]

    def colsel(xr, q2):
        if q2 == 0:
            return xr[:, :, 0:10, :]
        return jnp.concatenate([xr[:, :, 0:10, 64:], xr[:, :, 1:11, :64]],
                               axis=-1)

    m = [[None, None], [None, None]]
    for r2 in (0, 1):
        for q2 in (0, 1):
            p = jnp.concatenate(
                [colsel(rowsel(r2 + bi), q2) for bi in (0, 1)], axis=-1)
            a = jnp.dot(p.reshape(bb * 100, 256), w1_ref[...],
                        preferred_element_type=jnp.float32)
            a = jnp.maximum(a + b1_ref[...], 0.0).astype(jnp.bfloat16)
            m[r2][q2] = a.reshape(bb, 10, 10, 32)

    p = jnp.concatenate(
        [m[i % 2][j % 2][:, i // 2:i // 2 + 9, j // 2:j // 2 + 9, :]
         for i in range(4) for j in range(4)], axis=-1)
    a = jnp.dot(p.reshape(bb * 81, 512), w2_ref[...],
                preferred_element_type=jnp.float32)
    a = jnp.maximum(a + b2_ref[...], 0.0).astype(jnp.bfloat16)
    a = a.reshape(bb, 9, 9, 64)

    p = jnp.concatenate(
        [a[:, i:i + 7, j:j + 7, :] for i in range(3) for j in range(3)],
        axis=-1)
    a = jnp.dot(p.reshape(bb * 49, 576), w3_ref[...],
                preferred_element_type=jnp.float32)
    a = jnp.maximum(a + b3_ref[...], 0.0).astype(jnp.bfloat16)

    a = a.reshape(bb, 49, 64)
    flat = jnp.concatenate([a[:, p, :] for p in range(49)], axis=-1)

    h = jnp.dot(flat, wl1_ref[...], preferred_element_type=jnp.float32)
    h = jnp.maximum(h + bl1_ref[...], 0.0).astype(jnp.bfloat16)
    q = jnp.dot(h, wl2_ref[...], preferred_element_type=jnp.float32)
    o_ref[...] = q + bl2_ref[...]


def kernel(w1, b1, w2, b2, w3, b3, wl1, bl1, wl2, bl2, x):
    B = x.shape[0]
    Bp = (B + _BB - 1) // _BB * _BB
    if Bp != B:
        x = jnp.pad(x, ((0, Bp - B), (0, 0), (0, 0), (0, 0)))

    xp = jnp.pad(x, ((0, 0), (0, 0), (0, 4), (0, 4)))
    xs = jnp.transpose(xp.reshape(Bp, 4, 22, 4, 11, 8),
                       (0, 2, 4, 5, 1, 3)).reshape(Bp, 22, 11, 128)
    xs = xs.astype(jnp.bfloat16)
    w1p = w1[_PERM1]

    q = pl.pallas_call(
        _dqn_kernel,
        out_shape=jax.ShapeDtypeStruct((Bp, 128), jnp.float32),
        grid=(Bp // _BB,),
        in_specs=[
            pl.BlockSpec((_BB, 22, 11, 128), lambda i: (i, 0, 0, 0)),
            pl.BlockSpec((256, 32), lambda i: (0, 0)),
            pl.BlockSpec((1, 32), lambda i: (0, 0)),
            pl.BlockSpec((512, 64), lambda i: (0, 0)),
            pl.BlockSpec((1, 64), lambda i: (0, 0)),
            pl.BlockSpec((576, 64), lambda i: (0, 0)),
            pl.BlockSpec((1, 64), lambda i: (0, 0)),
            pl.BlockSpec((3136, 512), lambda i: (0, 0)),
            pl.BlockSpec((1, 512), lambda i: (0, 0)),
            pl.BlockSpec((512, 128), lambda i: (0, 0)),
            pl.BlockSpec((1, 128), lambda i: (0, 0)),
        ],
        out_specs=pl.BlockSpec((_BB, 128), lambda i: (i, 0)),
        compiler_params=pltpu.CompilerParams(
            dimension_semantics=("parallel",),
            vmem_limit_bytes=64 * 1024 * 1024,
        ),
    )(xs, w1p, b1.reshape(1, 32), w2, b2.reshape(1, 64), w3, b3.reshape(1, 64),
      wl1, bl1.reshape(1, 512), wl2, bl2.reshape(1, 128))
    return q[:B, :_N_ACT]


# speedup vs baseline: 1.7185x; 1.0151x over previous
import numpy as np

import jax
import jax.numpy as jnp
from jax.experimental import pallas as pl
from jax.experimental.pallas import tpu as pltpu

_N_ACT = 6
_BB = 64


def _w1_perm():
    perm = np.empty(256, np.int32)
    for bi in range(2):
        for j in range(8):
            for r in range(4):
                for c in range(4):
                    i = 4 * bi + r
                    perm[bi * 128 + j * 16 + c * 4 + r] = (i * 8 + j) * 4 + c
    return perm


_PERM1 = _w1_perm()


def _w2_perm():
    perm = np.empty(512, np.int32)
    for bi2 in range(2):
        for bj2 in range(2):
            for r2 in range(2):
                for q2 in range(2):
                    for c in range(32):
                        i, j = 2 * bi2 + r2, 2 * bj2 + q2
                        perm[(bi2 * 2 + bj2) * 128 + (r2 * 2 + q2) * 32 + c] = \
                            (i * 4 + j) * 32 + c
    return perm


_PERM2 = _w2_perm()


def _dqn_kernel(xs_ref, w1_ref, b1_ref, w2_ref, b2_ref, w3_ref, b3_ref,
                wl1_ref, bl1_ref, wl2_ref, bl2_ref, o_ref):
    bb = xs_ref.shape[0]
    x = xs_ref[...].reshape(bb, 11, 2, 11, 128)

    def rowsel(off):
        if off < 2:
            return x[:, 0:10, off]
        return x[:, 1:11, 0]

    def colsel(xr, q2):
        if q2 == 0:
            return xr[:, :, 0:10, :]
        return jnp.concatenate([xr[:, :, 0:10, 64:], xr[:, :, 1:11, :64]],
                               axis=-1)

    m = []
    for r2 in (0, 1):
        for q2 in (0, 1):
            p = jnp.concatenate(
                [colsel(rowsel(r2 + bi), q2) for bi in (0, 1)], axis=-1)
            a = jnp.dot(p.reshape(bb * 100, 256), w1_ref[...],
                        preferred_element_type=jnp.float32)
            a = jnp.maximum(a + b1_ref[...], 0.0).astype(jnp.bfloat16)
            m.append(a.reshape(bb, 10, 10, 32))
    mall = jnp.concatenate(m, axis=-1)

    acc = None
    for k, (bi2, bj2) in enumerate(((0, 0), (0, 1), (1, 0), (1, 1))):
        ps = mall[:, bi2:bi2 + 9, bj2:bj2 + 9, :]
        d = jnp.dot(ps.reshape(bb * 81, 128),
                    w2_ref[128 * k:128 * (k + 1), :],
                    preferred_element_type=jnp.float32)
        acc = d if acc is None else acc + d
    a = jnp.maximum(acc + b2_ref[...], 0.0).astype(jnp.bfloat16)
    a = a.reshape(bb, 9, 9, 64)

    acc = None
    for i in range(3):
        ps = jnp.concatenate([a[:, i:i + 7, j:j + 7, :] for j in range(3)],
                             axis=-1)
        d = jnp.dot(ps.reshape(bb * 49, 192),
                    w3_ref[192 * i:192 * (i + 1), :],
                    preferred_element_type=jnp.float32)
        acc = d if acc is None else acc + d
    a = jnp.maximum(acc + b3_ref[...], 0.0).astype(jnp.bfloat16)

    a = a.reshape(bb, 49, 64)
    flat = jnp.concatenate([a[:, p, :] for p in range(49)], axis=-1)

    h = jnp.dot(flat, wl1_ref[...], preferred_element_type=jnp.float32)
    h = jnp.maximum(h + bl1_ref[...], 0.0).astype(jnp.bfloat16)
    q = jnp.dot(h, wl2_ref[...], preferred_element_type=jnp.float32)
    o_ref[...] = q + bl2_ref[...]


def kernel(w1, b1, w2, b2, w3, b3, wl1, bl1, wl2, bl2, x):
    B = x.shape[0]
    Bp = (B + _BB - 1) // _BB * _BB
    if Bp != B:
        x = jnp.pad(x, ((0, Bp - B), (0, 0), (0, 0), (0, 0)))

    xp = jnp.pad(x, ((0, 0), (0, 0), (0, 4), (0, 4)))
    xs = jnp.transpose(xp.reshape(Bp, 4, 22, 4, 11, 8),
                       (0, 2, 4, 5, 1, 3)).reshape(Bp, 22, 11, 128)
    xs = xs.astype(jnp.bfloat16)
    w1p = w1[_PERM1]
    w2p = w2[_PERM2]

    q = pl.pallas_call(
        _dqn_kernel,
        out_shape=jax.ShapeDtypeStruct((Bp, 128), jnp.float32),
        grid=(Bp // _BB,),
        in_specs=[
            pl.BlockSpec((_BB, 22, 11, 128), lambda i: (i, 0, 0, 0)),
            pl.BlockSpec((256, 32), lambda i: (0, 0)),
            pl.BlockSpec((1, 32), lambda i: (0, 0)),
            pl.BlockSpec((512, 64), lambda i: (0, 0)),
            pl.BlockSpec((1, 64), lambda i: (0, 0)),
            pl.BlockSpec((576, 64), lambda i: (0, 0)),
            pl.BlockSpec((1, 64), lambda i: (0, 0)),
            pl.BlockSpec((3136, 512), lambda i: (0, 0)),
            pl.BlockSpec((1, 512), lambda i: (0, 0)),
            pl.BlockSpec((512, 128), lambda i: (0, 0)),
            pl.BlockSpec((1, 128), lambda i: (0, 0)),
        ],
        out_specs=pl.BlockSpec((_BB, 128), lambda i: (i, 0)),
        compiler_params=pltpu.CompilerParams(
            dimension_semantics=("parallel",),
            vmem_limit_bytes=64 * 1024 * 1024,
        ),
    )(xs, w1p, b1.reshape(1, 32), w2p, b2.reshape(1, 64), w3, b3.reshape(1, 64),
      wl1, bl1.reshape(1, 512), wl2, bl2.reshape(1, 128))
    return q[:B, :_N_ACT]


# speedup vs baseline: 1.9442x; 1.1313x over previous
import numpy as np

import jax
import jax.numpy as jnp
from jax.experimental import pallas as pl
from jax.experimental.pallas import tpu as pltpu

_N_ACT = 6
_BB = 64


def _w1_perm():
    perm = np.empty(256, np.int32)
    for bi in range(2):
        for j in range(8):
            for r in range(4):
                for c in range(4):
                    i = 4 * bi + r
                    perm[bi * 128 + j * 16 + c * 4 + r] = (i * 8 + j) * 4 + c
    return perm


_PERM1 = _w1_perm()


def _w2_perm():
    perm = np.empty(512, np.int32)
    for bi2 in range(2):
        for bj2 in range(2):
            for r2 in range(2):
                for q2 in range(2):
                    for c in range(32):
                        i, j = 2 * bi2 + r2, 2 * bj2 + q2
                        perm[(bi2 * 2 + bj2) * 128 + (r2 * 2 + q2) * 32 + c] = \
                            (i * 4 + j) * 32 + c
    return perm


_PERM2 = _w2_perm()


def _dqn_kernel(xs_ref, w1_ref, b1_ref, w2_ref, b2_ref, w3_ref, b3_ref,
                wl1_ref, bl1_ref, wl2_ref, bl2_ref, o_ref):
    bb = xs_ref.shape[0]

    def piece(off, q2):
        if off < 2:
            u, v = slice(0, 10), off
        else:
            u, v = slice(1, 11), 0
        if q2 == 0:
            return xs_ref[:, u, v, 0:10, :]
        return jnp.concatenate([xs_ref[:, u, v, 0:10, 64:],
                                xs_ref[:, u, v, 1:11, :64]], axis=-1)

    p = jnp.concatenate(
        [piece(r2 + bi, q2) for r2 in (0, 1) for q2 in (0, 1) for bi in (0, 1)],
        axis=-1)
    a = jnp.dot(p.reshape(bb * 100, 1024), w1_ref[...],
                preferred_element_type=jnp.float32)
    a = jnp.maximum(a + b1_ref[...], 0.0).astype(jnp.bfloat16)
    mall = a.reshape(bb, 10, 10, 128)

    acc = None
    for k, (bi2, bj2) in enumerate(((0, 0), (0, 1), (1, 0), (1, 1))):
        ps = mall[:, bi2:bi2 + 9, bj2:bj2 + 9, :]
        d = jnp.dot(ps.reshape(bb * 81, 128),
                    w2_ref[128 * k:128 * (k + 1), :],
                    preferred_element_type=jnp.float32)
        acc = d if acc is None else acc + d
    a = jnp.maximum(acc + b2_ref[...], 0.0).astype(jnp.bfloat16)
    a = a.reshape(bb, 9, 9, 64)

    acc = None
    for i in range(3):
        ps = jnp.concatenate([a[:, i:i + 7, j:j + 7, :] for j in range(3)],
                             axis=-1)
        d = jnp.dot(ps.reshape(bb * 49, 192),
                    w3_ref[192 * i:192 * (i + 1), :],
                    preferred_element_type=jnp.float32)
        acc = d if acc is None else acc + d
    a = jnp.maximum(acc + b3_ref[...], 0.0).astype(jnp.bfloat16)

    a = a.reshape(bb, 49, 64)
    flat = jnp.concatenate([a[:, p, :] for p in range(49)], axis=-1)

    h = jnp.dot(flat, wl1_ref[...], preferred_element_type=jnp.float32)
    h = jnp.maximum(h + bl1_ref[...], 0.0).astype(jnp.bfloat16)
    q = jnp.dot(h, wl2_ref[...], preferred_element_type=jnp.float32)
    o_ref[...] = q + bl2_ref[...]


def kernel(w1, b1, w2, b2, w3, b3, wl1, bl1, wl2, bl2, x):
    B = x.shape[0]
    Bp = (B + _BB - 1) // _BB * _BB
    if Bp != B:
        x = jnp.pad(x, ((0, Bp - B), (0, 0), (0, 0), (0, 0)))

    xp = jnp.pad(x, ((0, 0), (0, 0), (0, 4), (0, 4)))
    xs = jnp.transpose(xp.reshape(Bp, 4, 22, 4, 11, 8),
                       (0, 2, 4, 5, 1, 3)).reshape(Bp, 22, 11, 128)
    xs = xs.astype(jnp.bfloat16).reshape(Bp, 11, 2, 11, 128)
    w1p = w1[_PERM1]
    w1bd = jax.scipy.linalg.block_diag(w1p, w1p, w1p, w1p)
    b1t = jnp.tile(b1, 4)
    w2p = w2[_PERM2]

    q = pl.pallas_call(
        _dqn_kernel,
        out_shape=jax.ShapeDtypeStruct((Bp, 128), jnp.float32),
        grid=(Bp // _BB,),
        in_specs=[
            pl.BlockSpec((_BB, 11, 2, 11, 128), lambda i: (i, 0, 0, 0, 0)),
            pl.BlockSpec((1024, 128), lambda i: (0, 0)),
            pl.BlockSpec((1, 128), lambda i: (0, 0)),
            pl.BlockSpec((512, 64), lambda i: (0, 0)),
            pl.BlockSpec((1, 64), lambda i: (0, 0)),
            pl.BlockSpec((576, 64), lambda i: (0, 0)),
            pl.BlockSpec((1, 64), lambda i: (0, 0)),
            pl.BlockSpec((3136, 512), lambda i: (0, 0)),
            pl.BlockSpec((1, 512), lambda i: (0, 0)),
            pl.BlockSpec((512, 128), lambda i: (0, 0)),
            pl.BlockSpec((1, 128), lambda i: (0, 0)),
        ],
        out_specs=pl.BlockSpec((_BB, 128), lambda i: (i, 0)),
        compiler_params=pltpu.CompilerParams(
            dimension_semantics=("parallel",),
            vmem_limit_bytes=64 * 1024 * 1024,
        ),
    )(xs, w1bd, b1t.reshape(1, 128), w2p, b2.reshape(1, 64), w3, b3.reshape(1, 64),
      wl1, bl1.reshape(1, 512), wl2, bl2.reshape(1, 128))
    return q[:B, :_N_ACT]
